# baseline (speedup 1.0000x reference)
import functools

import jax
import jax.numpy as jnp
from jax import lax
from jax.experimental import pallas as pl
from jax.experimental.pallas import tpu as pltpu
from jax.experimental.pallas import tpu_sc as plsc

_N = 10000
_E = 320000
_D = 128
_G = 10
_NC = 2
_NS = 16
_NW = _NC * _NS
_EPT = _E // _NW

_K = 80
_CH = _EPT // _K

_mesh = plsc.VectorSubcoreMesh(core_axis_name="c", subcore_axis_name="s")


@functools.partial(
    pl.kernel,
    out_type=jax.ShapeDtypeStruct((_NC * _N, _D), jnp.float32),
    mesh=_mesh,
    scratch_types=[
        pltpu.VMEM_SHARED((_N, _D), jnp.float32),
        pltpu.VMEM((_EPT,), jnp.int32),
        pltpu.VMEM((_EPT,), jnp.int32),
        pltpu.VMEM((_K, _D), jnp.float32),
        pltpu.VMEM((_K, _D), jnp.float32),
        pltpu.VMEM((_K,), jnp.int32),
        pltpu.SemaphoreType.DMA,
        pltpu.SemaphoreType.DMA,
    ],
)
def _sc_agg(h_hbm, src_hbm, dst_hbm, out_hbm,
            acc, srcs, dsts, rows0, rows1, idxb, sem0, sem1):
    cid = lax.axis_index("c")
    sid = lax.axis_index("s")
    wid = sid * _NC + cid
    ebase = wid * _EPT

    pltpu.sync_copy(src_hbm.at[pl.ds(ebase, _EPT)], srcs)
    pltpu.sync_copy(dst_hbm.at[pl.ds(ebase, _EPT)], dsts)

    def _zrow(i, c):
        for j in range(_D // 16):
            rows0[i, pl.ds(j * 16, 16)] = jnp.zeros((16,), jnp.float32)
        return c
    lax.fori_loop(0, _K, _zrow, 0)

    @pl.when(sid < _G)
    def _zero_acc():
        r0 = sid * (_N // _G)
        for k in range(12):
            pltpu.sync_copy(rows0, acc.at[pl.ds(r0 + k * _K, _K), :])
        pltpu.sync_copy(rows0.at[pl.ds(0, 40), :],
                        acc.at[pl.ds(r0 + 960, 40), :])
    plsc.subcore_barrier()

    def _start(ci, rows, sem):
        pltpu.async_copy(h_hbm.at[srcs.at[pl.ds(ci * _K, _K)]], rows, sem)

    def _finish(ci, rows, sem):
        pltpu.make_async_copy(h_hbm.at[srcs.at[pl.ds(0, _K)]], rows,
                              sem).wait()
        for g in range(_K // 16):
            idxb[pl.ds(g * 16, 16)] = dsts[pl.ds(ci * _K + g * 16, 16)]
        pltpu.sync_copy(rows, acc.at[idxb], add=True)

    _start(0, rows0, sem0)

    def _pair(g, c):
        i0 = 2 * g
        _start(i0 + 1, rows1, sem1)
        _finish(i0, rows0, sem0)
        _start(i0 + 2, rows0, sem0)
        _finish(i0 + 1, rows1, sem1)
        return c
    lax.fori_loop(0, _CH // 2, _pair, 0)
    _finish(_CH - 1, rows0, sem0)

    plsc.subcore_barrier()

    @pl.when(sid < _G)
    def _copy_out():
        o0 = sid * (_N // _G)
        pltpu.sync_copy(acc.at[pl.ds(o0, _N // _G), :],
                        out_hbm.at[pl.ds(cid * _N + o0, _N // _G), :])


_BR = 1000


def _dense_relu(p, w, b):
    def body(p_ref, w_ref, b_ref, o_ref):
        agg = p_ref[0] + p_ref[1]
        o_ref[:] = jnp.maximum(
            jnp.dot(agg, w_ref[:], preferred_element_type=jnp.float32)
            + b_ref[:], 0.0)
    return pl.pallas_call(
        body,
        grid=(_N // _BR,),
        in_specs=[
            pl.BlockSpec((2, _BR, _D), lambda i: (0, i, 0)),
            pl.BlockSpec((_D, _D), lambda i: (0, 0)),
            pl.BlockSpec((1, _D), lambda i: (0, 0)),
        ],
        out_specs=pl.BlockSpec((_BR, _D), lambda i: (i, 0)),
        out_shape=jax.ShapeDtypeStruct((_N, _D), jnp.float32),
    )(p, w, b)


def _proj_readout(p, wf, bf, gs):
    def body(p_ref, wf_ref, bf_ref, gs_ref, o_ref):
        agg = p_ref[0] + p_ref[1]
        u = jnp.dot(agg, wf_ref[:],
                    preferred_element_type=jnp.float32) + bf_ref[0]
        s = jax.nn.sigmoid(u)
        i = pl.program_id(0)
        denom = gs_ref[i].astype(jnp.float32)
        o_ref[i] = jnp.sum(s) / denom
    return pl.pallas_call(
        body,
        grid=(_G,),
        in_specs=[
            pl.BlockSpec((2, _BR, _D), lambda i: (0, i, 0)),
            pl.BlockSpec((_D, 1), lambda i: (0, 0)),
            pl.BlockSpec(memory_space=pltpu.SMEM),
            pl.BlockSpec(memory_space=pltpu.SMEM),
        ],
        out_specs=pl.BlockSpec(memory_space=pltpu.SMEM),
        out_shape=jax.ShapeDtypeStruct((_G,), jnp.float32),
    )(p, wf, bf, gs)


def kernel(X, A, graph_sizes, W1, b1, W2, b2, Wf, bf):
    src = A[0]
    dst = A[1]

    p1 = _sc_agg(X, src, dst).reshape(_NC, _N, _D)
    h1 = _dense_relu(p1, W1, b1.reshape(1, _D))
    p2 = _sc_agg(h1, src, dst).reshape(_NC, _N, _D)
    h2 = _dense_relu(p2, W2, b2.reshape(1, _D))
    p3 = _sc_agg(h2, src, dst).reshape(_NC, _N, _D)
    return _proj_readout(p3, Wf, bf, graph_sizes)
